# baseline (speedup 1.0000x reference)
import functools

import jax
import jax.numpy as jnp
from jax import lax
from jax.experimental import pallas as pl

TILE_B = 256
IN_DIM = 1000
OUT_DIM = 1000


def _fused_body(x_ref, emb_ref, w1t_ref, b1_ref, w2t_ref, b2_ref, wfct_ref,
                out_ref):
    x = x_ref[:]
    iota = lax.broadcasted_iota(jnp.int32, x.shape, 1)
    m = jnp.max(x, axis=1, keepdims=True)
    idx = jnp.min(jnp.where(x == m, iota, IN_DIM), axis=1)
    oh = (iota == idx[:, None]).astype(jnp.float32)
    e = jnp.dot(oh, emb_ref[:], preferred_element_type=jnp.float32)
    h = jnp.maximum(
        jnp.dot(e, w1t_ref[:], preferred_element_type=jnp.float32)
        + b1_ref[:], 0.0)
    f = jnp.dot(h, w2t_ref[:], preferred_element_type=jnp.float32) + b2_ref[:]
    logits = jnp.dot(f, wfct_ref[:], preferred_element_type=jnp.float32)
    lm = jnp.max(logits, axis=1, keepdims=True)
    p = jnp.exp(logits - lm)
    out_ref[:] = p / jnp.sum(p, axis=1, keepdims=True)


@jax.jit
def kernel(x, emb, W1, b1, W2, b2, Wfc):
    batch, in_dim = x.shape
    hid = W1.shape[0]
    out_dim = Wfc.shape[0]
    grid = batch // TILE_B

    w1t = W1.T
    w2t = W2.T
    wfct = Wfc.T
    b1r = b1.reshape(1, hid)
    b2r = b2.reshape(1, hid)

    full = lambda shape: pl.BlockSpec(shape, lambda i: (0, 0))
    return pl.pallas_call(
        _fused_body,
        grid=(grid,),
        in_specs=[
            pl.BlockSpec((TILE_B, in_dim), lambda i: (i, 0)),
            full(emb.shape),
            full(w1t.shape),
            full(b1r.shape),
            full(w2t.shape),
            full(b2r.shape),
            full(wfct.shape),
        ],
        out_specs=pl.BlockSpec((TILE_B, out_dim), lambda i: (i, 0)),
        out_shape=jax.ShapeDtypeStruct((batch, out_dim), jnp.float32),
    )(x, emb, w1t, b1r, w2t, b2r, wfct)


# speedup vs baseline: 2.0792x; 2.0792x over previous
import jax
import jax.numpy as jnp
from jax import lax
from jax.experimental import pallas as pl

TILE_B = 256


def _fused_body(xt_ref, embt_ref, w1_ref, b1_ref, w2_ref, b2_ref, wfc_ref,
                out_ref):
    xt = xt_ref[:]
    in_dim = xt.shape[0]
    iota = lax.broadcasted_iota(jnp.int32, xt.shape, 0)
    m = jnp.max(xt, axis=0, keepdims=True)
    idx = jnp.min(jnp.where(xt == m, iota, in_dim), axis=0, keepdims=True)
    oh = (iota == idx).astype(jnp.float32)
    e = jnp.dot(embt_ref[:], oh, preferred_element_type=jnp.float32)
    h = jnp.maximum(
        jnp.dot(w1_ref[:], e, preferred_element_type=jnp.float32)
        + b1_ref[:], 0.0)
    f = jnp.dot(w2_ref[:], h, preferred_element_type=jnp.float32) + b2_ref[:]
    logits = jnp.dot(wfc_ref[:], f, preferred_element_type=jnp.float32)
    lm = jnp.max(logits, axis=0, keepdims=True)
    p = jnp.exp(logits - lm)
    out_ref[:] = p / jnp.sum(p, axis=0, keepdims=True)


@jax.jit
def kernel(x, emb, W1, b1, W2, b2, Wfc):
    batch, in_dim = x.shape
    hid = W1.shape[0]
    out_dim = Wfc.shape[0]
    grid = batch // TILE_B

    xt = x.T
    embt = emb.T
    b1c = b1.reshape(hid, 1)
    b2c = b2.reshape(hid, 1)

    full = lambda shape: pl.BlockSpec(shape, lambda i: (0, 0))
    outt = pl.pallas_call(
        _fused_body,
        grid=(grid,),
        in_specs=[
            pl.BlockSpec((in_dim, TILE_B), lambda i: (0, i)),
            full(embt.shape),
            full(W1.shape),
            full(b1c.shape),
            full(W2.shape),
            full(b2c.shape),
            full(Wfc.shape),
        ],
        out_specs=pl.BlockSpec((out_dim, TILE_B), lambda i: (0, i)),
        out_shape=jax.ShapeDtypeStruct((out_dim, batch), jnp.float32),
    )(xt, embt, W1, b1c, W2, b2c, Wfc)
    return outt.T


# speedup vs baseline: 3.5655x; 1.7148x over previous
import jax
import jax.numpy as jnp
from jax import lax
from jax.experimental import pallas as pl

TILE_B = 1024


def _fused_body(xt_ref, embt_ref, w1_ref, b1_ref, w2_ref, b2_ref, wfc_ref,
                out_ref):
    xt = xt_ref[:]
    in_dim = xt.shape[0]
    iota = lax.broadcasted_iota(jnp.int32, xt.shape, 0)
    m = jnp.max(xt, axis=0, keepdims=True)
    idx = jnp.min(jnp.where(xt == m, iota, in_dim), axis=0, keepdims=True)
    oh = (iota == idx).astype(jnp.float32)
    e = jnp.dot(embt_ref[:], oh, preferred_element_type=jnp.float32)
    h = jnp.maximum(
        jnp.dot(w1_ref[:], e, preferred_element_type=jnp.float32)
        + b1_ref[:], 0.0)
    f = jnp.dot(w2_ref[:], h, preferred_element_type=jnp.float32) + b2_ref[:]
    logits = jnp.dot(wfc_ref[:], f, preferred_element_type=jnp.float32)
    lm = jnp.max(logits, axis=0, keepdims=True)
    p = jnp.exp(logits - lm)
    out_ref[:] = p / jnp.sum(p, axis=0, keepdims=True)


@jax.jit
def kernel(x, emb, W1, b1, W2, b2, Wfc):
    batch, in_dim = x.shape
    hid = W1.shape[0]
    out_dim = Wfc.shape[0]
    grid = batch // TILE_B

    xt = x.T
    embt = emb.T
    b1c = b1.reshape(hid, 1)
    b2c = b2.reshape(hid, 1)

    full = lambda shape: pl.BlockSpec(shape, lambda i: (0, 0))
    outt = pl.pallas_call(
        _fused_body,
        grid=(grid,),
        in_specs=[
            pl.BlockSpec((in_dim, TILE_B), lambda i: (0, i)),
            full(embt.shape),
            full(W1.shape),
            full(b1c.shape),
            full(W2.shape),
            full(b2c.shape),
            full(Wfc.shape),
        ],
        out_specs=pl.BlockSpec((out_dim, TILE_B), lambda i: (0, i)),
        out_shape=jax.ShapeDtypeStruct((out_dim, batch), jnp.float32),
    )(xt, embt, W1, b1c, W2, b2c, Wfc)
    return outt.T


# speedup vs baseline: 3.7706x; 1.0575x over previous
import jax
import jax.numpy as jnp
from jax import lax
from jax.experimental import pallas as pl

TILE_B = 2048


def _fused_body(xt_ref, embt_ref, w1_ref, b1_ref, w2_ref, b2_ref, wfc_ref,
                out_ref):
    xt = xt_ref[:]
    in_dim = xt.shape[0]
    iota = lax.broadcasted_iota(jnp.int32, xt.shape, 0)
    m = jnp.max(xt, axis=0, keepdims=True)
    idx = jnp.min(jnp.where(xt == m, iota, in_dim), axis=0, keepdims=True)
    oh = (iota == idx).astype(jnp.float32)
    e = jnp.dot(embt_ref[:], oh, preferred_element_type=jnp.float32)
    h = jnp.maximum(
        jnp.dot(w1_ref[:], e, preferred_element_type=jnp.float32)
        + b1_ref[:], 0.0)
    f = jnp.dot(w2_ref[:], h, preferred_element_type=jnp.float32) + b2_ref[:]
    logits = jnp.dot(wfc_ref[:], f, preferred_element_type=jnp.float32)
    lm = jnp.max(logits, axis=0, keepdims=True)
    p = jnp.exp(logits - lm)
    out_ref[:] = p / jnp.sum(p, axis=0, keepdims=True)


@jax.jit
def kernel(x, emb, W1, b1, W2, b2, Wfc):
    batch, in_dim = x.shape
    hid = W1.shape[0]
    out_dim = Wfc.shape[0]
    grid = batch // TILE_B

    xt = x.T
    embt = emb.T
    b1c = b1.reshape(hid, 1)
    b2c = b2.reshape(hid, 1)

    full = lambda shape: pl.BlockSpec(shape, lambda i: (0, 0))
    outt = pl.pallas_call(
        _fused_body,
        grid=(grid,),
        in_specs=[
            pl.BlockSpec((in_dim, TILE_B), lambda i: (0, i)),
            full(embt.shape),
            full(W1.shape),
            full(b1c.shape),
            full(W2.shape),
            full(b2c.shape),
            full(Wfc.shape),
        ],
        out_specs=pl.BlockSpec((out_dim, TILE_B), lambda i: (0, i)),
        out_shape=jax.ShapeDtypeStruct((out_dim, batch), jnp.float32),
    )(xt, embt, W1, b1c, W2, b2c, Wfc)
    return outt.T
